# Initial kernel scaffold; baseline (speedup 1.0000x reference)
import jax
import jax.numpy as jnp
from jax.experimental import pallas as pl


def kernel(admission_type, admission_location, drg_type, drg_severity, drg_mortality, emb_admission_type, emb_admission_location, emb_drg_type, emb_drg_severity, emb_drg_mortality, W1, b1, ln_g, ln_b, W2, b2):
    raise NotImplementedError("write your pallas kernel here")


# speedup vs baseline: 1.0440x; 1.0440x over previous
import functools

import jax
import jax.numpy as jnp
from jax import lax
from jax.experimental import pallas as pl
from jax.experimental.pallas import tpu as pltpu
from jax.experimental.pallas import tpu_sc as plsc

_B = 16384
_NF = 5
_ED = 8
_NW = 32
_ROWS = _NF * _B
_RPW = _ROWS // _NW
_CH = 128
_NCH = _RPW // _CH

_TC_BS = 1024


def _sc_gather(idx3d, tcat):
    mesh = plsc.VectorSubcoreMesh(core_axis_name="c", subcore_axis_name="s")

    @functools.partial(
        pl.kernel,
        mesh=mesh,
        out_type=jax.ShapeDtypeStruct((_ROWS, _ED), jnp.float32),
        scratch_types=[
            pltpu.VMEM((_NCH, _CH), jnp.int32),
            pltpu.VMEM((_RPW, _ED), jnp.float32),
            pltpu.SemaphoreType.DMA,
        ],
        compiler_params=pltpu.CompilerParams(use_tc_tiling_on_sc=False),
    )
    def k(idx_hbm, tab_hbm, out_hbm, idx_v, rows_v, sem):
        wid = lax.axis_index("s") * 2 + lax.axis_index("c")
        pltpu.sync_copy(idx_hbm.at[wid], idx_v)
        copies = [
            pltpu.async_copy(
                tab_hbm.at[idx_v.at[j]], rows_v.at[pl.ds(j * _CH, _CH)], sem
            )
            for j in range(_NCH)
        ]
        for c in copies:
            c.wait()
        pltpu.sync_copy(rows_v, out_hbm.at[pl.ds(wid * _RPW, _RPW)])

    return k(idx3d, tcat)


def _tc_mlp(x, W1, b1, ln_g, ln_b, W2, b2):
    nb = _B // _TC_BS

    def body(x_ref, w1_ref, b1_ref, g_ref, be_ref, w2_ref, b2_ref, o_ref):
        h = jnp.dot(x_ref[0], w1_ref[0:_ED, :],
                    preferred_element_type=jnp.float32)
        for f in range(1, _NF):
            h = h + jnp.dot(x_ref[f], w1_ref[f * _ED:(f + 1) * _ED, :],
                            preferred_element_type=jnp.float32)
        h = h + b1_ref[...]
        mu = jnp.mean(h, axis=1, keepdims=True)
        var = jnp.mean((h - mu) ** 2, axis=1, keepdims=True)
        h = (h - mu) * lax.rsqrt(var + 1e-5) * g_ref[...] + be_ref[...]
        h = jnp.maximum(h, 0.0)
        o_ref[...] = (
            jnp.dot(h, w2_ref[...], preferred_element_type=jnp.float32)
            + b2_ref[...]
        )

    return pl.pallas_call(
        body,
        grid=(nb,),
        in_specs=[
            pl.BlockSpec((_NF, _TC_BS, _ED), lambda i: (0, i, 0)),
            pl.BlockSpec((_NF * _ED, 128), lambda i: (0, 0)),
            pl.BlockSpec((1, 128), lambda i: (0, 0)),
            pl.BlockSpec((1, 128), lambda i: (0, 0)),
            pl.BlockSpec((1, 128), lambda i: (0, 0)),
            pl.BlockSpec((128, 64), lambda i: (0, 0)),
            pl.BlockSpec((1, 64), lambda i: (0, 0)),
        ],
        out_specs=pl.BlockSpec((_TC_BS, 64), lambda i: (i, 0)),
        out_shape=jax.ShapeDtypeStruct((_B, 64), jnp.float32),
    )(x, W1, b1, ln_g, ln_b, W2, b2)


def kernel(admission_type, admission_location, drg_type, drg_severity,
           drg_mortality, emb_admission_type, emb_admission_location,
           emb_drg_type, emb_drg_severity, emb_drg_mortality,
           W1, b1, ln_g, ln_b, W2, b2):
    idxs = [admission_type, admission_location, drg_type, drg_severity,
            drg_mortality]
    tabs = [emb_admission_type, emb_admission_location, emb_drg_type,
            emb_drg_severity, emb_drg_mortality]
    offs = []
    off = 0
    for t in tabs:
        offs.append(off)
        off += t.shape[0]
    tcat = jnp.concatenate(tabs, axis=0)
    tcat = jnp.pad(tcat, ((0, (-off) % 8), (0, 0)))
    idxcat = jnp.concatenate(
        [i.astype(jnp.int32) + o for i, o in zip(idxs, offs)])
    idx3d = idxcat.reshape(_NW, _NCH, _CH)

    x = _sc_gather(idx3d, tcat).reshape(_NF, _B, _ED)
    return _tc_mlp(
        x, W1,
        b1.reshape(1, 128), ln_g.reshape(1, 128), ln_b.reshape(1, 128),
        W2, b2.reshape(1, 64),
    )


# speedup vs baseline: 4.5976x; 4.4040x over previous
import functools

import jax
import jax.numpy as jnp
from jax import lax
from jax.experimental import pallas as pl
from jax.experimental.pallas import tpu as pltpu
from jax.experimental.pallas import tpu_sc as plsc

_B = 16384
_NF = 5
_ED = 8
_NW = 32
_ROWS = _NF * _B
_RPW = _ROWS // _NW
_NG = _RPW // 16

_TC_BS = 1024


def _sc_gather(idx2d, tflat):
    mesh = plsc.VectorSubcoreMesh(core_axis_name="c", subcore_axis_name="s")

    @functools.partial(
        pl.kernel,
        mesh=mesh,
        out_type=jax.ShapeDtypeStruct((_ED, _ROWS), jnp.float32),
        scratch_types=[
            pltpu.VMEM((_RPW,), jnp.int32),
            pltpu.VMEM((320,), jnp.float32),
            pltpu.VMEM((_ED, _RPW), jnp.float32),
        ],
        compiler_params=pltpu.CompilerParams(
            use_tc_tiling_on_sc=False, needs_layout_passes=False),
    )
    def k(idx_hbm, tab_hbm, out_hbm, idx_v, tab_v, cols_v):
        wid = lax.axis_index("s") * 2 + lax.axis_index("c")
        pltpu.sync_copy(idx_hbm.at[wid], idx_v)
        pltpu.sync_copy(tab_hbm, tab_v)

        def body(i, _):
            idx8 = idx_v[pl.ds(i * 16, 16)] * 8
            for c in range(_ED):
                vals = plsc.load_gather(tab_v, [idx8 + c])
                cols_v[c, pl.ds(i * 16, 16)] = vals
            return _

        lax.fori_loop(0, _NG, body, None)
        for c in range(_ED):
            pltpu.sync_copy(
                cols_v.at[pl.ds(c, 1)],
                out_hbm.at[pl.ds(c, 1), pl.ds(wid * _RPW, _RPW)])

    return k(idx2d, tflat)


def _tc_mlp(xT, W1, b1, ln_g, ln_b, W2, b2):
    nb = _B // _TC_BS
    dn = (((0,), (0,)), ((), ()))

    def body(x0, x1, x2, x3, x4, w1_ref, b1_ref, g_ref, be_ref, w2_ref,
             b2_ref, o_ref):
        xs = (x0, x1, x2, x3, x4)
        h = lax.dot_general(xs[0][...], w1_ref[0:_ED, :], dn,
                            preferred_element_type=jnp.float32)
        for f in range(1, _NF):
            h = h + lax.dot_general(
                xs[f][...], w1_ref[f * _ED:(f + 1) * _ED, :], dn,
                preferred_element_type=jnp.float32)
        h = h + b1_ref[...]
        mu = jnp.mean(h, axis=1, keepdims=True)
        var = jnp.mean((h - mu) ** 2, axis=1, keepdims=True)
        h = (h - mu) * lax.rsqrt(var + 1e-5) * g_ref[...] + be_ref[...]
        h = jnp.maximum(h, 0.0)
        o_ref[...] = (
            jnp.dot(h, w2_ref[...], preferred_element_type=jnp.float32)
            + b2_ref[...]
        )

    return pl.pallas_call(
        body,
        grid=(nb,),
        in_specs=[
            pl.BlockSpec((_ED, _TC_BS), lambda i, f=f: (0, f * nb + i))
            for f in range(_NF)
        ] + [
            pl.BlockSpec((_NF * _ED, 128), lambda i: (0, 0)),
            pl.BlockSpec((1, 128), lambda i: (0, 0)),
            pl.BlockSpec((1, 128), lambda i: (0, 0)),
            pl.BlockSpec((1, 128), lambda i: (0, 0)),
            pl.BlockSpec((128, 64), lambda i: (0, 0)),
            pl.BlockSpec((1, 64), lambda i: (0, 0)),
        ],
        out_specs=pl.BlockSpec((_TC_BS, 64), lambda i: (i, 0)),
        out_shape=jax.ShapeDtypeStruct((_B, 64), jnp.float32),
    )(xT, xT, xT, xT, xT, W1, b1, ln_g, ln_b, W2, b2)


def kernel(admission_type, admission_location, drg_type, drg_severity,
           drg_mortality, emb_admission_type, emb_admission_location,
           emb_drg_type, emb_drg_severity, emb_drg_mortality,
           W1, b1, ln_g, ln_b, W2, b2):
    idxs = [admission_type, admission_location, drg_type, drg_severity,
            drg_mortality]
    tabs = [emb_admission_type, emb_admission_location, emb_drg_type,
            emb_drg_severity, emb_drg_mortality]
    offs = []
    off = 0
    for t in tabs:
        offs.append(off)
        off += t.shape[0]
    tflat = jnp.concatenate(tabs, axis=0).reshape(-1)
    tflat = jnp.pad(tflat, (0, 320 - tflat.shape[0]))
    idxcat = jnp.concatenate(
        [i.astype(jnp.int32) + o for i, o in zip(idxs, offs)])
    idx2d = idxcat.reshape(_NW, _RPW)

    xT = _sc_gather(idx2d, tflat)
    return _tc_mlp(
        xT, W1,
        b1.reshape(1, 128), ln_g.reshape(1, 128), ln_b.reshape(1, 128),
        W2, b2.reshape(1, 64),
    )


# speedup vs baseline: 5.4014x; 1.1748x over previous
import functools

import jax
import jax.numpy as jnp
from jax import lax
from jax.experimental import pallas as pl
from jax.experimental.pallas import tpu as pltpu
from jax.experimental.pallas import tpu_sc as plsc

_B = 16384
_NF = 5
_ED = 8
_NW = 32
_BPW = _B // _NW
_GPF = _BPW // 16

_TC_BS = 1024


def _sc_gather(idxs, tabs):
    mesh = plsc.VectorSubcoreMesh(core_axis_name="c", subcore_axis_name="s")
    sizes = [t.shape[0] for t in tabs]
    offs = [sum(sizes[:f]) for f in range(_NF)]

    @functools.partial(
        pl.kernel,
        mesh=mesh,
        out_type=jax.ShapeDtypeStruct((_ED, _NF * _B), jnp.float32),
        scratch_types=[
            pltpu.VMEM((_NF * _BPW,), jnp.int32),
            pltpu.VMEM((40, _ED), jnp.float32),
            pltpu.VMEM((_ED, _NF * _BPW), jnp.float32),
            pltpu.SemaphoreType.DMA,
        ],
        compiler_params=pltpu.CompilerParams(
            use_tc_tiling_on_sc=False, needs_layout_passes=False),
    )
    def k(i0, i1, i2, i3, i4, t0, t1, t2, t3, t4, out_hbm,
          idx_v, tab_v, cols_v, sem):
        wid = lax.axis_index("s") * 2 + lax.axis_index("c")
        base = wid * _BPW
        stage = [
            pltpu.async_copy(ih.at[pl.ds(base, _BPW)],
                             idx_v.at[pl.ds(f * _BPW, _BPW)], sem)
            for f, ih in enumerate((i0, i1, i2, i3, i4))
        ] + [
            pltpu.async_copy(th, tab_v.at[pl.ds(offs[f], sizes[f])], sem)
            for f, th in enumerate((t0, t1, t2, t3, t4))
        ]
        for cp in stage:
            cp.wait()

        cvecs = [jnp.full((16,), c, jnp.int32) for c in range(_ED)]

        for f in range(_NF):

            def body_f(i, _, f=f):
                rows = idx_v[pl.ds(f * _BPW + i * 16, 16)] + offs[f]
                for c in range(_ED):
                    vals = plsc.load_gather(tab_v, [rows, cvecs[c]])
                    cols_v[c, pl.ds(f * _BPW + i * 16, 16)] = vals
                return _

            lax.fori_loop(0, _GPF, body_f, None)

        drain = [
            pltpu.async_copy(
                cols_v.at[pl.ds(c, 1), pl.ds(f * _BPW, _BPW)],
                out_hbm.at[pl.ds(c, 1), pl.ds(f * _B + base, _BPW)], sem)
            for c in range(_ED) for f in range(_NF)
        ]
        for cp in drain:
            cp.wait()

    return k(*idxs, *tabs)


def _tc_mlp(xT, W1, b1, ln_g, ln_b, W2, b2):
    nb = _B // _TC_BS
    dn = (((0,), (0,)), ((), ()))

    def body(x0, x1, x2, x3, x4, w1_ref, b1_ref, g_ref, be_ref, w2_ref,
             b2_ref, o_ref):
        xs = (x0, x1, x2, x3, x4)
        xcat = jnp.concatenate([x[...] for x in xs], axis=0)
        h = lax.dot_general(xcat, w1_ref[...], dn,
                            preferred_element_type=jnp.float32)
        h = h + b1_ref[...]
        mu = jnp.mean(h, axis=1, keepdims=True)
        var = jnp.mean((h - mu) ** 2, axis=1, keepdims=True)
        h = (h - mu) * lax.rsqrt(var + 1e-5) * g_ref[...] + be_ref[...]
        h = jnp.maximum(h, 0.0)
        o_ref[...] = (
            jnp.dot(h, w2_ref[...], preferred_element_type=jnp.float32)
            + b2_ref[...]
        )

    return pl.pallas_call(
        body,
        grid=(nb,),
        in_specs=[
            pl.BlockSpec((_ED, _TC_BS), lambda i, f=f: (0, f * nb + i))
            for f in range(_NF)
        ] + [
            pl.BlockSpec((_NF * _ED, 128), lambda i: (0, 0)),
            pl.BlockSpec((1, 128), lambda i: (0, 0)),
            pl.BlockSpec((1, 128), lambda i: (0, 0)),
            pl.BlockSpec((1, 128), lambda i: (0, 0)),
            pl.BlockSpec((128, 64), lambda i: (0, 0)),
            pl.BlockSpec((1, 64), lambda i: (0, 0)),
        ],
        out_specs=pl.BlockSpec((_TC_BS, 64), lambda i: (i, 0)),
        out_shape=jax.ShapeDtypeStruct((_B, 64), jnp.float32),
    )(xT, xT, xT, xT, xT, W1, b1, ln_g, ln_b, W2, b2)


def kernel(admission_type, admission_location, drg_type, drg_severity,
           drg_mortality, emb_admission_type, emb_admission_location,
           emb_drg_type, emb_drg_severity, emb_drg_mortality,
           W1, b1, ln_g, ln_b, W2, b2):
    idxs = [admission_type.astype(jnp.int32),
            admission_location.astype(jnp.int32),
            drg_type.astype(jnp.int32),
            drg_severity.astype(jnp.int32),
            drg_mortality.astype(jnp.int32)]
    tabs = [emb_admission_type, emb_admission_location, emb_drg_type,
            emb_drg_severity, emb_drg_mortality]

    xT = _sc_gather(idxs, tabs)
    return _tc_mlp(
        xT, W1,
        b1.reshape(1, 128), ln_g.reshape(1, 128), ln_b.reshape(1, 128),
        W2, b2.reshape(1, 64),
    )


# speedup vs baseline: 5.6462x; 1.0453x over previous
import functools

import jax
import jax.numpy as jnp
from jax import lax
from jax.experimental import pallas as pl
from jax.experimental.pallas import tpu as pltpu
from jax.experimental.pallas import tpu_sc as plsc

_B = 16384
_NF = 5
_ED = 8
_Q = _NF * _ED
_NW = 32
_BPW = _B // _NW
_HPW = _BPW // 128
_RPW = _HPW * _Q

_TC_BS = 1024
_TC_RB = _TC_BS // 128 * _Q


def _sc_gather(idxs, tabs):
    mesh = plsc.VectorSubcoreMesh(core_axis_name="c", subcore_axis_name="s")
    sizes = [t.shape[0] for t in tabs]
    offs = [sum(sizes[:f]) for f in range(_NF)]

    @functools.partial(
        pl.kernel,
        mesh=mesh,
        out_type=jax.ShapeDtypeStruct((_B // 128 * _Q, 128), jnp.float32),
        scratch_types=[
            pltpu.VMEM((_NF * _BPW,), jnp.int32),
            pltpu.VMEM((40, _ED), jnp.float32),
            pltpu.VMEM((_RPW, 128), jnp.float32),
            pltpu.SemaphoreType.DMA,
        ],
        compiler_params=pltpu.CompilerParams(
            use_tc_tiling_on_sc=False, needs_layout_passes=False),
    )
    def k(i0, i1, i2, i3, i4, t0, t1, t2, t3, t4, out_hbm,
          idx_v, tab_v, buf_v, sem):
        wid = lax.axis_index("s") * 2 + lax.axis_index("c")
        base = wid * _BPW
        stage = [
            pltpu.async_copy(ih.at[pl.ds(base, _BPW)],
                             idx_v.at[pl.ds(f * _BPW, _BPW)], sem)
            for f, ih in enumerate((i0, i1, i2, i3, i4))
        ] + [
            pltpu.async_copy(th, tab_v.at[pl.ds(offs[f], sizes[f])], sem)
            for f, th in enumerate((t0, t1, t2, t3, t4))
        ]
        for cp in stage:
            cp.wait()

        cvecs = [jnp.full((16,), c, jnp.int32) for c in range(_ED)]

        for f in range(_NF):
            for hi in range(_HPW):

                def body(p, _, f=f, hi=hi):
                    rows = idx_v[pl.ds(f * _BPW + hi * 128 + p * 16, 16)]
                    rows = rows + offs[f]
                    for c in range(_ED):
                        vals = plsc.load_gather(tab_v, [rows, cvecs[c]])
                        buf_v[hi * _Q + f * _ED + c, pl.ds(p * 16, 16)] = vals
                    return _

                lax.fori_loop(0, 8, body, None)

        pltpu.async_copy(buf_v, out_hbm.at[pl.ds(wid * _RPW, _RPW)],
                         sem).wait()

    return k(*idxs, *tabs)


def _tc_mlp(xq, W1, b1, ln_g, ln_b, W2, b2):
    nb = _B // _TC_BS
    dn = (((0,), (0,)), ((), ()))

    def body(x_ref, w1_ref, b1_ref, g_ref, be_ref, w2_ref, b2_ref, o_ref):
        hs = [
            lax.dot_general(x_ref[pl.ds(k * _Q, _Q), :], w1_ref[...], dn,
                            preferred_element_type=jnp.float32)
            for k in range(_TC_BS // 128)
        ]
        h = jnp.concatenate(hs, axis=0)
        h = h + b1_ref[...]
        mu = jnp.mean(h, axis=1, keepdims=True)
        var = jnp.mean((h - mu) ** 2, axis=1, keepdims=True)
        h = (h - mu) * lax.rsqrt(var + 1e-5) * g_ref[...] + be_ref[...]
        h = jnp.maximum(h, 0.0)
        o_ref[...] = (
            jnp.dot(h, w2_ref[...], preferred_element_type=jnp.float32)
            + b2_ref[...]
        )

    return pl.pallas_call(
        body,
        grid=(nb,),
        in_specs=[
            pl.BlockSpec((_TC_RB, 128), lambda i: (i, 0)),
            pl.BlockSpec((_Q, 128), lambda i: (0, 0)),
            pl.BlockSpec((128,), lambda i: (0,)),
            pl.BlockSpec((128,), lambda i: (0,)),
            pl.BlockSpec((128,), lambda i: (0,)),
            pl.BlockSpec((128, 64), lambda i: (0, 0)),
            pl.BlockSpec((64,), lambda i: (0,)),
        ],
        out_specs=pl.BlockSpec((_TC_BS, 64), lambda i: (i, 0)),
        out_shape=jax.ShapeDtypeStruct((_B, 64), jnp.float32),
    )(xq, W1, b1, ln_g, ln_b, W2, b2)


def kernel(admission_type, admission_location, drg_type, drg_severity,
           drg_mortality, emb_admission_type, emb_admission_location,
           emb_drg_type, emb_drg_severity, emb_drg_mortality,
           W1, b1, ln_g, ln_b, W2, b2):
    idxs = [admission_type.astype(jnp.int32),
            admission_location.astype(jnp.int32),
            drg_type.astype(jnp.int32),
            drg_severity.astype(jnp.int32),
            drg_mortality.astype(jnp.int32)]
    tabs = [emb_admission_type, emb_admission_location, emb_drg_type,
            emb_drg_severity, emb_drg_mortality]

    xq = _sc_gather(idxs, tabs)
    return _tc_mlp(xq, W1, b1, ln_g, ln_b, W2, b2)
